# Initial kernel scaffold; baseline (speedup 1.0000x reference)
import jax
import jax.numpy as jnp
from jax.experimental import pallas as pl


def kernel(x, router_logits, w1, w3, w2):
    raise NotImplementedError("write your pallas kernel here")


# speedup vs baseline: 1.8512x; 1.8512x over previous
import jax
import jax.numpy as jnp
from jax.experimental import pallas as pl
from jax.experimental.pallas import tpu as pltpu

TOKENS = 256
HIDDEN = 1024
NUM_EXPERTS = 16
FF = 2048
TOP_K = 2


def _moe_kernel(x_ref, rl_ref, w1_ref, w3_ref, w2_ref, out_ref, combine_ref):
    e = pl.program_id(0)

    @pl.when(e == 0)
    def _():
        logits = rl_ref[...]
        mx = jnp.max(logits, axis=-1, keepdims=True)
        ex = jnp.exp(logits - mx)
        p = ex / jnp.sum(ex, axis=-1, keepdims=True)
        eidx = jax.lax.broadcasted_iota(jnp.int32, p.shape, 1)
        m1 = jnp.max(p, axis=-1, keepdims=True)
        i1 = jnp.min(jnp.where(p == m1, eidx, NUM_EXPERTS), axis=-1, keepdims=True)
        p2 = jnp.where(eidx == i1, -1.0, p)
        m2 = jnp.max(p2, axis=-1, keepdims=True)
        i2 = jnp.min(jnp.where(p2 == m2, eidx, NUM_EXPERTS), axis=-1, keepdims=True)
        s = m1 + m2
        combine_ref[...] = jnp.where(eidx == i1, m1 / s, 0.0) + jnp.where(
            eidx == i2, m2 / s, 0.0
        )

    xv = x_ref[...]
    h1 = jnp.dot(xv, w1_ref[0], preferred_element_type=jnp.float32)
    h3 = jnp.dot(xv, w3_ref[0], preferred_element_type=jnp.float32)
    act = (h1 * jax.lax.logistic(h1) * h3).astype(jnp.bfloat16)
    y = jnp.dot(act, w2_ref[0], preferred_element_type=jnp.float32)

    lane = jax.lax.broadcasted_iota(jnp.int32, (TOKENS, NUM_EXPERTS), 1)
    wcol = jnp.sum(
        jnp.where(lane == e, combine_ref[...], 0.0), axis=-1, keepdims=True
    )

    @pl.when(e == 0)
    def _():
        out_ref[...] = wcol * y

    @pl.when(e != 0)
    def _():
        out_ref[...] += wcol * y


def kernel(x, router_logits, w1, w3, w2):
    return pl.pallas_call(
        _moe_kernel,
        grid=(NUM_EXPERTS,),
        in_specs=[
            pl.BlockSpec((TOKENS, HIDDEN), lambda e: (0, 0)),
            pl.BlockSpec((TOKENS, NUM_EXPERTS), lambda e: (0, 0)),
            pl.BlockSpec((1, HIDDEN, FF), lambda e: (e, 0, 0)),
            pl.BlockSpec((1, HIDDEN, FF), lambda e: (e, 0, 0)),
            pl.BlockSpec((1, FF, HIDDEN), lambda e: (e, 0, 0)),
        ],
        out_specs=pl.BlockSpec((TOKENS, HIDDEN), lambda e: (0, 0)),
        out_shape=jax.ShapeDtypeStruct((TOKENS, HIDDEN), jnp.float32),
        scratch_shapes=[pltpu.VMEM((TOKENS, NUM_EXPERTS), jnp.float32)],
        compiler_params=pltpu.CompilerParams(
            dimension_semantics=("arbitrary",),
        ),
    )(x, router_logits, w1, w3, w2)
